# baseline (speedup 1.0000x reference)
import jax
import jax.numpy as jnp
from jax import lax
from jax.experimental import pallas as pl
from jax.experimental.pallas import tpu as pltpu
from jax.experimental.pallas import tpu_sc as plsc

NUM_CLASSES = 40
TABLE_ROWS = NUM_CLASSES + 1
OUT_CHANNELS = 512
N = 100000

LANES = 16
NUM_WORKERS = 32
CHUNK = 80
NUM_CHUNKS = N // CHUNK
CHUNKS_PER_WORKER = -(-NUM_CHUNKS // NUM_WORKERS)
GROUPS = CHUNK // LANES
TABLE_WORDS = TABLE_ROWS * OUT_CHANNELS
CHUNK_WORDS = CHUNK * OUT_CHANNELS
UNROLL = 4


def _sc_body(y_hbm, mask_hbm, table_hbm, out_hbm, *refs):
    (y_v, m_v, idx_v, rows_a, rows_b, table_v, ysem, wsem) = refs
    rows_bufs = (rows_a, rows_b)
    nc = plsc.get_sparse_core_info().num_cores
    wid = lax.axis_index("s") * nc + lax.axis_index("c")

    pltpu.sync_copy(table_hbm.at[pl.ds(wid * TABLE_WORDS, TABLE_WORDS)], table_v)

    def chunk_id(t):
        return wid + t * NUM_WORKERS

    def load_start(t, p):
        base = chunk_id(t) * CHUNK
        pltpu.async_copy(y_hbm.at[pl.ds(base, CHUNK)], y_v.at[p], ysem.at[p])
        pltpu.async_copy(mask_hbm.at[pl.ds(base, CHUNK)], m_v.at[p], ysem.at[p])

    def load_wait(t, p):
        base = chunk_id(t) * CHUNK
        pltpu.make_async_copy(y_hbm.at[pl.ds(base, CHUNK)], y_v.at[p], ysem.at[p]).wait()
        pltpu.make_async_copy(mask_hbm.at[pl.ds(base, CHUNK)], m_v.at[p], ysem.at[p]).wait()

    def writeout_wait(t, p):
        base = chunk_id(t) * CHUNK_WORDS
        pltpu.make_async_copy(
            rows_bufs[p], out_hbm.at[pl.ds(base, CHUNK_WORDS)], wsem.at[p]
        ).wait()

    lane_iota = jax.lax.iota(jnp.int32, LANES)

    def process(t, p):
        valid = chunk_id(t) < NUM_CHUNKS

        @pl.when(valid)
        def _():
            load_wait(t, p)
            for j in range(GROUPS):
                sl = pl.ds(j * LANES, LANES)
                idx_v[p, sl] = jnp.where(m_v[p, sl] != 0, y_v[p, sl], NUM_CLASSES)

            @pl.when(t >= 2)
            def _():
                writeout_wait(t - 2, p)

            @pl.when(chunk_id(t + 2) < NUM_CHUNKS)
            def _():
                load_start(t + 2, p)

            row_off = [idx_v[p, pl.ds(g * LANES, LANES)] * OUT_CHANNELS for g in range(GROUPS)]
            node_off = [(lane_iota + g * LANES) * OUT_CHANNELS for g in range(GROUPS)]
            rows_p = rows_bufs[p]

            def chan_step(i, chan):
                for _ in range(UNROLL):
                    for g in range(GROUPS):
                        val = plsc.load_gather(table_v, [row_off[g] + chan])
                        plsc.store_scatter(rows_p, [node_off[g] + chan], val)
                    chan = chan + 1
                return chan

            lax.fori_loop(
                0, OUT_CHANNELS // UNROLL, chan_step,
                jnp.zeros((LANES,), jnp.int32),
            )

            base = chunk_id(t) * CHUNK_WORDS
            pltpu.async_copy(rows_p, out_hbm.at[pl.ds(base, CHUNK_WORDS)], wsem.at[p])

    load_start(0, 0)
    load_start(1, 1)

    def pair_step(i, carry):
        process(2 * i, 0)
        process(2 * i + 1, 1)
        return carry

    lax.fori_loop(0, CHUNKS_PER_WORKER // 2, pair_step, 0)

    for t in (CHUNKS_PER_WORKER - 2, CHUNKS_PER_WORKER - 1):
        @pl.when(chunk_id(t) < NUM_CHUNKS)
        def _(t=t):
            writeout_wait(t, t % 2)


@jax.jit
def _masked_lookup(y, mask_i32, table_flat):
    mesh = plsc.VectorSubcoreMesh(core_axis_name="c", subcore_axis_name="s")
    out_flat = pl.kernel(
        _sc_body,
        out_type=jax.ShapeDtypeStruct((N * OUT_CHANNELS,), jnp.float32),
        mesh=mesh,
        compiler_params=pltpu.CompilerParams(needs_layout_passes=False),
        scratch_types=[
            pltpu.VMEM((2, CHUNK), jnp.int32),
            pltpu.VMEM((2, CHUNK), jnp.int32),
            pltpu.VMEM((2, CHUNK), jnp.int32),
            pltpu.VMEM((CHUNK_WORDS,), jnp.float32),
            pltpu.VMEM((CHUNK_WORDS,), jnp.float32),
            pltpu.VMEM((TABLE_WORDS,), jnp.float32),
            pltpu.SemaphoreType.DMA((2,)),
            pltpu.SemaphoreType.DMA((2,)),
        ],
    )(y, mask_i32, table_flat)
    return out_flat.reshape(N, OUT_CHANNELS)


def kernel(y, mask, emb):
    table = jnp.concatenate(
        [emb, jnp.zeros((1, OUT_CHANNELS), dtype=emb.dtype)], axis=0
    )
    table_flat = jnp.tile(table.reshape(-1), NUM_WORKERS)
    return _masked_lookup(y.astype(jnp.int32), mask.astype(jnp.int32), table_flat)


# speedup vs baseline: 2.2465x; 2.2465x over previous
import jax
import jax.numpy as jnp
from jax import lax
from jax.experimental import pallas as pl
from jax.experimental.pallas import tpu as pltpu
from jax.experimental.pallas import tpu_sc as plsc

NUM_CLASSES = 40
TABLE_ROWS = NUM_CLASSES + 1
OUT_CHANNELS = 512
N = 100000

LANES = 16
NUM_WORKERS = 32
CHUNK = 80
NUM_CHUNKS = N // CHUNK
CHUNKS_PER_WORKER = -(-NUM_CHUNKS // NUM_WORKERS)
GROUPS = CHUNK // LANES
TABLE_WORDS = TABLE_ROWS * OUT_CHANNELS
CHUNK_WORDS = CHUNK * OUT_CHANNELS
UNROLL = 4


def _sc_body(y_hbm, mask_hbm, table_hbm, out_hbm, *refs):
    (y_v, m_v, idx_v, rows_a, rows_b, table_v, ysem, wsem) = refs
    rows_bufs = (rows_a, rows_b)
    nc = plsc.get_sparse_core_info().num_cores
    wid = lax.axis_index("s") * nc + lax.axis_index("c")

    pltpu.sync_copy(table_hbm.at[pl.ds(wid * TABLE_WORDS, TABLE_WORDS)], table_v)

    def chunk_id(t):
        return wid + t * NUM_WORKERS

    def load_start(t, p):
        base = chunk_id(t) * CHUNK
        pltpu.async_copy(y_hbm.at[pl.ds(base, CHUNK)], y_v.at[p], ysem.at[p])
        pltpu.async_copy(mask_hbm.at[pl.ds(base, CHUNK)], m_v.at[p], ysem.at[p])

    def load_wait(t, p):
        base = chunk_id(t) * CHUNK
        pltpu.make_async_copy(y_hbm.at[pl.ds(base, CHUNK)], y_v.at[p], ysem.at[p]).wait()
        pltpu.make_async_copy(mask_hbm.at[pl.ds(base, CHUNK)], m_v.at[p], ysem.at[p]).wait()

    def writeout_wait(t, p):
        base = chunk_id(t) * CHUNK_WORDS
        pltpu.make_async_copy(
            rows_bufs[p], out_hbm.at[pl.ds(base, CHUNK_WORDS)], wsem.at[p]
        ).wait()

    lane_iota = jax.lax.iota(jnp.int32, LANES)

    def process(t, p):
        valid = chunk_id(t) < NUM_CHUNKS

        @pl.when(valid)
        def _():
            load_wait(t, p)
            for j in range(GROUPS):
                sl = pl.ds(j * LANES, LANES)
                idx_v[p, sl] = jnp.where(m_v[p, sl] != 0, y_v[p, sl], NUM_CLASSES)

            @pl.when(t >= 2)
            def _():
                writeout_wait(t - 2, p)

            @pl.when(chunk_id(t + 2) < NUM_CHUNKS)
            def _():
                load_start(t + 2, p)

            row_off = [idx_v[p, pl.ds(g * LANES, LANES)] * OUT_CHANNELS for g in range(GROUPS)]
            node_off = [(lane_iota + g * LANES) * OUT_CHANNELS for g in range(GROUPS)]
            rows_p = rows_bufs[p]

            zeros16 = jnp.zeros((LANES,), jnp.int32)

            @plsc.parallel_loop(0, OUT_CHANNELS, step=1, unroll=UNROLL)
            def _(c):
                chan = zeros16 + c
                for g in range(GROUPS):
                    val = plsc.load_gather(table_v, [row_off[g] + chan])
                    plsc.store_scatter(rows_p, [node_off[g] + chan], val)

            base = chunk_id(t) * CHUNK_WORDS
            pltpu.async_copy(rows_p, out_hbm.at[pl.ds(base, CHUNK_WORDS)], wsem.at[p])

    load_start(0, 0)
    load_start(1, 1)

    def pair_step(i, carry):
        process(2 * i, 0)
        process(2 * i + 1, 1)
        return carry

    lax.fori_loop(0, CHUNKS_PER_WORKER // 2, pair_step, 0)

    for t in (CHUNKS_PER_WORKER - 2, CHUNKS_PER_WORKER - 1):
        @pl.when(chunk_id(t) < NUM_CHUNKS)
        def _(t=t):
            writeout_wait(t, t % 2)


@jax.jit
def _masked_lookup(y, mask_i32, table_flat):
    mesh = plsc.VectorSubcoreMesh(core_axis_name="c", subcore_axis_name="s")
    out_flat = pl.kernel(
        _sc_body,
        out_type=jax.ShapeDtypeStruct((N * OUT_CHANNELS,), jnp.float32),
        mesh=mesh,
        compiler_params=pltpu.CompilerParams(needs_layout_passes=False),
        scratch_types=[
            pltpu.VMEM((2, CHUNK), jnp.int32),
            pltpu.VMEM((2, CHUNK), jnp.int32),
            pltpu.VMEM((2, CHUNK), jnp.int32),
            pltpu.VMEM((CHUNK_WORDS,), jnp.float32),
            pltpu.VMEM((CHUNK_WORDS,), jnp.float32),
            pltpu.VMEM((TABLE_WORDS,), jnp.float32),
            pltpu.SemaphoreType.DMA((2,)),
            pltpu.SemaphoreType.DMA((2,)),
        ],
    )(y, mask_i32, table_flat)
    return out_flat.reshape(N, OUT_CHANNELS)


def kernel(y, mask, emb):
    table = jnp.concatenate(
        [emb, jnp.zeros((1, OUT_CHANNELS), dtype=emb.dtype)], axis=0
    )
    table_flat = jnp.tile(table.reshape(-1), NUM_WORKERS)
    return _masked_lookup(y.astype(jnp.int32), mask.astype(jnp.int32), table_flat)


# speedup vs baseline: 7.2068x; 3.2081x over previous
import jax
import jax.numpy as jnp
from jax import lax
from jax.experimental import pallas as pl
from jax.experimental.pallas import tpu as pltpu
from jax.experimental.pallas import tpu_sc as plsc

NUM_CLASSES = 40
TABLE_ROWS = NUM_CLASSES + 1
OUT_CHANNELS = 512
N = 100000

LANES = 16
NUM_WORKERS = 32
CHUNK = 80
NUM_CHUNKS = N // CHUNK
CHUNKS_PER_WORKER = -(-NUM_CHUNKS // NUM_WORKERS)
GROUPS = CHUNK // LANES
TABLE_WORDS = TABLE_ROWS * OUT_CHANNELS
CHUNK_WORDS = CHUNK * OUT_CHANNELS
UNROLL = 4


def _sc_body(y_hbm, mask_hbm, table_hbm, out_hbm, *refs):
    (y_v, m_v, idx_a, idx_b, rows_a, rows_b, table_v, ysem, wsem) = refs
    rows_bufs = (rows_a, rows_b)
    idx_bufs = (idx_a, idx_b)
    nc = plsc.get_sparse_core_info().num_cores
    wid = lax.axis_index("s") * nc + lax.axis_index("c")

    pltpu.sync_copy(table_hbm.at[pl.ds(wid * TABLE_WORDS, TABLE_WORDS)], table_v)

    def chunk_id(t):
        return wid + t * NUM_WORKERS

    def load_start(t, p):
        base = chunk_id(t) * CHUNK
        pltpu.async_copy(y_hbm.at[pl.ds(base, CHUNK)], y_v.at[p], ysem.at[p])
        pltpu.async_copy(mask_hbm.at[pl.ds(base, CHUNK)], m_v.at[p], ysem.at[p])

    def load_wait(t, p):
        base = chunk_id(t) * CHUNK
        pltpu.make_async_copy(y_hbm.at[pl.ds(base, CHUNK)], y_v.at[p], ysem.at[p]).wait()
        pltpu.make_async_copy(mask_hbm.at[pl.ds(base, CHUNK)], m_v.at[p], ysem.at[p]).wait()

    def writeout_wait(t, p):
        base = chunk_id(t) * CHUNK_WORDS
        pltpu.make_async_copy(
            rows_bufs[p], out_hbm.at[pl.ds(base, CHUNK_WORDS)], wsem.at[p]
        ).wait()

    lane_iota = jax.lax.iota(jnp.int32, LANES)

    def process(t, p):
        valid = chunk_id(t) < NUM_CHUNKS

        @pl.when(valid)
        def _():
            load_wait(t, p)
            for j in range(GROUPS):
                sl = pl.ds(j * LANES, LANES)
                idx_bufs[p][sl] = (
                    jnp.where(m_v[p, sl] != 0, y_v[p, sl], NUM_CLASSES)
                    * OUT_CHANNELS
                )

            @pl.when(t >= 2)
            def _():
                writeout_wait(t - 2, p)

            @pl.when(chunk_id(t + 2) < NUM_CHUNKS)
            def _():
                load_start(t + 2, p)

            rows_p = rows_bufs[p]
            idx_p = idx_bufs[p]
            zeros16 = jnp.zeros((LANES,), jnp.int32)

            @plsc.parallel_loop(0, CHUNK, step=1, unroll=UNROLL)
            def _(n):
                row16 = plsc.load_gather(idx_p, [zeros16 + n])
                src = row16 + lane_iota
                for j in range(OUT_CHANNELS // LANES):
                    val = plsc.load_gather(
                        table_v.at[pl.ds(j * LANES, TABLE_WORDS - j * LANES)],
                        [src],
                    )
                    rows_p[pl.ds(n * OUT_CHANNELS + j * LANES, LANES)] = val

            base = chunk_id(t) * CHUNK_WORDS
            pltpu.async_copy(rows_p, out_hbm.at[pl.ds(base, CHUNK_WORDS)], wsem.at[p])

    load_start(0, 0)
    load_start(1, 1)

    def pair_step(i, carry):
        process(2 * i, 0)
        process(2 * i + 1, 1)
        return carry

    lax.fori_loop(0, CHUNKS_PER_WORKER // 2, pair_step, 0)

    for t in (CHUNKS_PER_WORKER - 2, CHUNKS_PER_WORKER - 1):
        @pl.when(chunk_id(t) < NUM_CHUNKS)
        def _(t=t):
            writeout_wait(t, t % 2)


@jax.jit
def _masked_lookup(y, mask_i32, table_flat):
    mesh = plsc.VectorSubcoreMesh(core_axis_name="c", subcore_axis_name="s")
    out_flat = pl.kernel(
        _sc_body,
        out_type=jax.ShapeDtypeStruct((N * OUT_CHANNELS,), jnp.float32),
        mesh=mesh,
        compiler_params=pltpu.CompilerParams(needs_layout_passes=False),
        scratch_types=[
            pltpu.VMEM((2, CHUNK), jnp.int32),
            pltpu.VMEM((2, CHUNK), jnp.int32),
            pltpu.VMEM((CHUNK,), jnp.int32),
            pltpu.VMEM((CHUNK,), jnp.int32),
            pltpu.VMEM((CHUNK_WORDS,), jnp.float32),
            pltpu.VMEM((CHUNK_WORDS,), jnp.float32),
            pltpu.VMEM((TABLE_WORDS,), jnp.float32),
            pltpu.SemaphoreType.DMA((2,)),
            pltpu.SemaphoreType.DMA((2,)),
        ],
    )(y, mask_i32, table_flat)
    return out_flat.reshape(N, OUT_CHANNELS)


def kernel(y, mask, emb):
    table = jnp.concatenate(
        [emb, jnp.zeros((1, OUT_CHANNELS), dtype=emb.dtype)], axis=0
    )
    table_flat = jnp.tile(table.reshape(-1), NUM_WORKERS)
    return _masked_lookup(y.astype(jnp.int32), mask.astype(jnp.int32), table_flat)


# speedup vs baseline: 7.2295x; 1.0031x over previous
import jax
import jax.numpy as jnp
from jax import lax
from jax.experimental import pallas as pl
from jax.experimental.pallas import tpu as pltpu
from jax.experimental.pallas import tpu_sc as plsc

NUM_CLASSES = 40
TABLE_ROWS = NUM_CLASSES + 1
OUT_CHANNELS = 512
N = 100000

LANES = 16
NUM_WORKERS = 32
CHUNK = 80
NUM_CHUNKS = N // CHUNK
CHUNKS_PER_WORKER = -(-NUM_CHUNKS // NUM_WORKERS)
GROUPS = CHUNK // LANES
TABLE_WORDS = TABLE_ROWS * OUT_CHANNELS
CHUNK_WORDS = CHUNK * OUT_CHANNELS
UNROLL = 8


def _sc_body(y_hbm, mask_hbm, table_hbm, out_hbm, *refs):
    (y_v, m_v, idx_a, idx_b, rows_a, rows_b, table_v, ysem, wsem) = refs
    rows_bufs = (rows_a, rows_b)
    idx_bufs = (idx_a, idx_b)
    nc = plsc.get_sparse_core_info().num_cores
    wid = lax.axis_index("s") * nc + lax.axis_index("c")

    pltpu.sync_copy(table_hbm.at[pl.ds(wid * TABLE_WORDS, TABLE_WORDS)], table_v)

    def chunk_id(t):
        return wid + t * NUM_WORKERS

    def load_start(t, p):
        base = chunk_id(t) * CHUNK
        pltpu.async_copy(y_hbm.at[pl.ds(base, CHUNK)], y_v.at[p], ysem.at[p])
        pltpu.async_copy(mask_hbm.at[pl.ds(base, CHUNK)], m_v.at[p], ysem.at[p])

    def load_wait(t, p):
        base = chunk_id(t) * CHUNK
        pltpu.make_async_copy(y_hbm.at[pl.ds(base, CHUNK)], y_v.at[p], ysem.at[p]).wait()
        pltpu.make_async_copy(mask_hbm.at[pl.ds(base, CHUNK)], m_v.at[p], ysem.at[p]).wait()

    def writeout_wait(t, p):
        base = chunk_id(t) * CHUNK_WORDS
        pltpu.make_async_copy(
            rows_bufs[p], out_hbm.at[pl.ds(base, CHUNK_WORDS)], wsem.at[p]
        ).wait()

    lane_iota = jax.lax.iota(jnp.int32, LANES)

    def process(t, p):
        valid = chunk_id(t) < NUM_CHUNKS

        @pl.when(valid)
        def _():
            load_wait(t, p)
            for j in range(GROUPS):
                sl = pl.ds(j * LANES, LANES)
                idx_bufs[p][sl] = (
                    jnp.where(m_v[p, sl] != 0, y_v[p, sl], NUM_CLASSES)
                    * OUT_CHANNELS
                )

            @pl.when(t >= 2)
            def _():
                writeout_wait(t - 2, p)

            @pl.when(chunk_id(t + 2) < NUM_CHUNKS)
            def _():
                load_start(t + 2, p)

            rows_p = rows_bufs[p]
            idx_p = idx_bufs[p]
            zeros16 = jnp.zeros((LANES,), jnp.int32)

            @plsc.parallel_loop(0, CHUNK, step=1, unroll=UNROLL)
            def _(n):
                row16 = plsc.load_gather(idx_p, [zeros16 + n])
                src = row16 + lane_iota
                for j in range(OUT_CHANNELS // LANES):
                    val = plsc.load_gather(
                        table_v.at[pl.ds(j * LANES, TABLE_WORDS - j * LANES)],
                        [src],
                    )
                    rows_p[pl.ds(n * OUT_CHANNELS + j * LANES, LANES)] = val

            base = chunk_id(t) * CHUNK_WORDS
            pltpu.async_copy(rows_p, out_hbm.at[pl.ds(base, CHUNK_WORDS)], wsem.at[p])

    load_start(0, 0)
    load_start(1, 1)

    def pair_step(i, carry):
        process(2 * i, 0)
        process(2 * i + 1, 1)
        return carry

    lax.fori_loop(0, CHUNKS_PER_WORKER // 2, pair_step, 0)

    for t in (CHUNKS_PER_WORKER - 2, CHUNKS_PER_WORKER - 1):
        @pl.when(chunk_id(t) < NUM_CHUNKS)
        def _(t=t):
            writeout_wait(t, t % 2)


@jax.jit
def _masked_lookup(y, mask_i32, table_flat):
    mesh = plsc.VectorSubcoreMesh(core_axis_name="c", subcore_axis_name="s")
    out_flat = pl.kernel(
        _sc_body,
        out_type=jax.ShapeDtypeStruct((N * OUT_CHANNELS,), jnp.float32),
        mesh=mesh,
        compiler_params=pltpu.CompilerParams(needs_layout_passes=False),
        scratch_types=[
            pltpu.VMEM((2, CHUNK), jnp.int32),
            pltpu.VMEM((2, CHUNK), jnp.int32),
            pltpu.VMEM((CHUNK,), jnp.int32),
            pltpu.VMEM((CHUNK,), jnp.int32),
            pltpu.VMEM((CHUNK_WORDS,), jnp.float32),
            pltpu.VMEM((CHUNK_WORDS,), jnp.float32),
            pltpu.VMEM((TABLE_WORDS,), jnp.float32),
            pltpu.SemaphoreType.DMA((2,)),
            pltpu.SemaphoreType.DMA((2,)),
        ],
    )(y, mask_i32, table_flat)
    return out_flat.reshape(N, OUT_CHANNELS)


def kernel(y, mask, emb):
    table = jnp.concatenate(
        [emb, jnp.zeros((1, OUT_CHANNELS), dtype=emb.dtype)], axis=0
    )
    table_flat = jnp.tile(table.reshape(-1), NUM_WORKERS)
    return _masked_lookup(y.astype(jnp.int32), mask.astype(jnp.int32), table_flat)
